# Initial kernel scaffold; baseline (speedup 1.0000x reference)
import jax
import jax.numpy as jnp
from jax.experimental import pallas as pl


def kernel(x, edge_index, node_type, edge_type, edge_attr, hetero_W, hetero_b, edge_type_emb_table, W_edge_attr, W_att, W_lin, b_lin):
    raise NotImplementedError("write your pallas kernel here")


# speedup vs baseline: 4.3316x; 4.3316x over previous
import functools

import jax
import jax.numpy as jnp
from jax import lax
from jax.experimental import pallas as pl
from jax.experimental.pallas import tpu as pltpu
from jax.experimental.pallas import tpu_sc as plsc

N = 10000
NP = 10240
E = 320000
D = 128
CH = 64
NCH = E // CH
NWORK = 32
HALF = NP // 2
KMAX = -(-NCH // 16)
BN = 2048
BE = 8000


def _node_body(x_ref, nt_ref, hw_ref, hb_ref, wl_ref, wai_ref, waj_ref,
               xh_ref, m_ref, sd_ref, ss_ref):
    xb = x_ref[...]
    nt = nt_ref[...]
    acc = jnp.zeros((BN, D), dtype=jnp.float32)
    for t in range(3):
        pt = jnp.dot(xb, hw_ref[t], preferred_element_type=jnp.float32)
        pt = pt + hb_ref[pl.ds(t, 1), :]
        acc = jnp.where(nt == t, pt, acc)
    xh_ref[...] = acc
    m_ref[...] = jnp.dot(acc, wl_ref[...], preferred_element_type=jnp.float32)
    sd_ref[...] = jnp.dot(acc, wai_ref[...], preferred_element_type=jnp.float32)
    ss_ref[...] = jnp.dot(acc, waj_ref[...], preferred_element_type=jnp.float32)


def _node_stage(x_p, nt_p, hetero_W, hetero_b, wl_top, wa_i, wa_j):
    grid = (NP // BN,)
    return pl.pallas_call(
        _node_body,
        grid=grid,
        in_specs=[
            pl.BlockSpec((BN, D), lambda i: (i, 0)),
            pl.BlockSpec((BN, 1), lambda i: (i, 0)),
            pl.BlockSpec((3, D, D), lambda i: (0, 0, 0)),
            pl.BlockSpec((3, D), lambda i: (0, 0)),
            pl.BlockSpec((D, D), lambda i: (0, 0)),
            pl.BlockSpec((D, 1), lambda i: (0, 0)),
            pl.BlockSpec((D, 1), lambda i: (0, 0)),
        ],
        out_specs=[
            pl.BlockSpec((BN, D), lambda i: (i, 0)),
            pl.BlockSpec((BN, D), lambda i: (i, 0)),
            pl.BlockSpec((BN, 1), lambda i: (i, 0)),
            pl.BlockSpec((BN, 1), lambda i: (i, 0)),
        ],
        out_shape=[
            jax.ShapeDtypeStruct((NP, D), jnp.float32),
            jax.ShapeDtypeStruct((NP, D), jnp.float32),
            jax.ShapeDtypeStruct((NP, 1), jnp.float32),
            jax.ShapeDtypeStruct((NP, 1), jnp.float32),
        ],
    )(x_p, nt_p, hetero_W, hetero_b, wl_top, wa_i, wa_j)


def _edge_body(eattr_ref, et_ref, wea_ref, waa_ref, t5_ref, ea_ref, c_ref):
    v = jnp.dot(eattr_ref[...], wea_ref[...], preferred_element_type=jnp.float32)
    eab = jnp.where(v >= 0, v, 0.2 * v)
    ea_ref[...] = eab
    cc = jnp.dot(eab, waa_ref[...], preferred_element_type=jnp.float32)
    et = et_ref[...]
    for t in range(5):
        cc = jnp.where(et == t, cc + t5_ref[t], cc)
    c_ref[...] = cc


def _edge_stage(edge_attr, et_p, W_ea, wa_a, t5_pad):
    grid = (E // BE,)
    return pl.pallas_call(
        _edge_body,
        grid=grid,
        in_specs=[
            pl.BlockSpec((BE, 16), lambda i: (i, 0)),
            pl.BlockSpec((BE, 1), lambda i: (i, 0)),
            pl.BlockSpec((16, 16), lambda i: (0, 0)),
            pl.BlockSpec((16, 1), lambda i: (0, 0)),
            pl.BlockSpec(memory_space=pltpu.SMEM),
        ],
        out_specs=[
            pl.BlockSpec((BE, 16), lambda i: (i, 0)),
            pl.BlockSpec((BE, 1), lambda i: (i, 0)),
        ],
        out_shape=[
            jax.ShapeDtypeStruct((E, 16), jnp.float32),
            jax.ShapeDtypeStruct((E, 1), jnp.float32),
        ],
    )(edge_attr, et_p, W_ea, wa_a, t5_pad)


def _sc_body(src_h, dst_h, c_h, ea_h, m_h, sd_h, ss_h,
             outm_o, agg_o, den_o,
             sd_v, ss_v, den_v, src_v, dst_v, c_v, ex_v, sg_v, dl_v,
             m_v, ea_v,
             outm_sh, agg_sh, sem):
    cid = lax.axis_index("c")
    sid = lax.axis_index("s")
    wid = sid * 2 + cid
    lo = cid * HALF
    z16v = jnp.zeros((16,), jnp.float32)

    def zrow(i, _):
        for kk in range(D // 16):
            m_v[i, pl.ds(kk * 16, 16)] = z16v
        ea_v[i, :] = z16v
        return 0

    lax.fori_loop(0, CH, zrow, 0)

    def zden(i, _):
        den_v[i, :] = z16v
        return 0

    lax.fori_loop(0, NP // 16, zden, 0)

    rpm = HALF // 16
    for j in range(rpm // CH):
        r0 = sid * rpm + j * CH
        pltpu.sync_copy(m_v, outm_sh.at[pl.ds(r0, CH), :])
        pltpu.sync_copy(ea_v, agg_sh.at[pl.ds(r0, CH), :])
    pltpu.sync_copy(sd_h, sd_v)
    pltpu.sync_copy(ss_h, ss_v)
    plsc.subcore_barrier()

    def chunk_body(k, _):
        ci = sid + k * 16

        @pl.when(ci < NCH)
        def _():
            base = ci * CH
            pltpu.sync_copy(src_h.at[pl.ds(base, CH)], src_v)
            pltpu.sync_copy(dst_h.at[pl.ds(base, CH)], dst_v)
            pltpu.sync_copy(c_h.at[pl.ds(base, CH)], c_v)
            for g in range(CH // 16):
                sl = pl.ds(g * 16, 16)
                s16 = src_v[sl]
                d16 = dst_v[sl]
                dloc = d16 - lo
                inh = (dloc >= 0) & (dloc < HALF)
                neg1 = jnp.zeros((16,), jnp.int32) - 1
                sg_v[sl] = jnp.where(inh, s16, neg1)
                dl_v[sl] = jnp.where(inh, dloc, neg1)
                z = (plsc.load_gather(sd_v, [d16]) +
                     plsc.load_gather(ss_v, [s16]) + c_v[sl])
                z = jnp.where(z >= 0, z, 0.2 * z)
                ex = jnp.exp(z)
                ex_v[sl] = ex
                plsc.addupdate_scatter(
                    den_v, [lax.shift_right_logical(d16, 4),
                            lax.bitwise_and(d16, 15)], ex, mask=inh)
            cp = pltpu.async_copy(
                m_h.at[plsc.Indices(sg_v, ignored_value=-1)], m_v, sem)
            pltpu.sync_copy(ea_h.at[pl.ds(base, CH), :], ea_v)
            cp.wait()

            def row_fn(i, _):
                s = plsc.load_gather(ex_v, [jnp.zeros((16,), jnp.int32) + i])
                for kk in range(D // 16):
                    rsl = pl.ds(kk * 16, 16)
                    m_v[i, rsl] = m_v[i, rsl] * s
                ea_v[i, :] = ea_v[i, :] * s
                return 0

            lax.fori_loop(0, CH, row_fn, 0)
            pltpu.sync_copy(
                m_v, outm_sh.at[plsc.Indices(dl_v, ignored_value=-1)],
                add=True)
            pltpu.sync_copy(
                ea_v, agg_sh.at[plsc.Indices(dl_v, ignored_value=-1)],
                add=True)
        return 0

    lax.fori_loop(0, KMAX, chunk_body, 0)

    pltpu.sync_copy(den_v, den_o.at[wid])
    plsc.subcore_barrier()

    for j in range(rpm // CH):
        r0 = sid * rpm + j * CH
        pltpu.sync_copy(outm_sh.at[pl.ds(r0, CH), :], m_v)
        pltpu.sync_copy(m_v, outm_o.at[cid, pl.ds(r0, CH), :])
        pltpu.sync_copy(agg_sh.at[pl.ds(r0, CH), :], ea_v)
        pltpu.sync_copy(ea_v, agg_o.at[cid, pl.ds(r0, CH), :])


def _sc_stage(srcA, dstA, cA, ea, m, sd, ss):
    mesh = plsc.VectorSubcoreMesh(core_axis_name="c", subcore_axis_name="s")
    fn = pl.kernel(
        _sc_body,
        out_type=[
            jax.ShapeDtypeStruct((2, HALF, D), jnp.float32),
            jax.ShapeDtypeStruct((2, HALF, 16), jnp.float32),
            jax.ShapeDtypeStruct((NWORK, NP // 16, 16), jnp.float32),
        ],
        mesh=mesh,
        compiler_params=pltpu.CompilerParams(
            needs_layout_passes=False, use_tc_tiling_on_sc=False),
        scratch_types=[
            pltpu.VMEM((NP,), jnp.float32),
            pltpu.VMEM((NP,), jnp.float32),
            pltpu.VMEM((NP // 16, 16), jnp.float32),
            pltpu.VMEM((CH,), jnp.int32),
            pltpu.VMEM((CH,), jnp.int32),
            pltpu.VMEM((CH,), jnp.float32),
            pltpu.VMEM((CH,), jnp.float32),
            pltpu.VMEM((CH,), jnp.int32),
            pltpu.VMEM((CH,), jnp.int32),
            pltpu.VMEM((CH, D), jnp.float32),
            pltpu.VMEM((CH, 16), jnp.float32),
            pltpu.VMEM_SHARED((HALF, D), jnp.float32),
            pltpu.VMEM_SHARED((HALF, 16), jnp.float32),
            pltpu.SemaphoreType.DMA,
        ],
    )
    return fn(srcA, dstA, cA, ea, m, sd, ss)


def _den_body(den_ref, inv_ref, suma_ref):
    dsum = jnp.sum(den_ref[...], axis=0)
    inv = 1.0 / (dsum + 1e-16)
    inv_ref[...] = inv
    suma_ref[...] = dsum * inv


def _den_stage(den_p):
    return pl.pallas_call(
        _den_body,
        out_shape=[
            jax.ShapeDtypeStruct((NP // 16, 16), jnp.float32),
            jax.ShapeDtypeStruct((NP // 16, 16), jnp.float32),
        ],
    )(den_p)


def _comb_body(outm_ref, agg_ref, xh_ref, inv_ref, suma_ref, wb_ref, bl_ref,
               out_ref):
    core = outm_ref[...] + jnp.dot(
        agg_ref[...], wb_ref[...], preferred_element_type=jnp.float32)
    out_ref[...] = (core * inv_ref[...] + suma_ref[...] * bl_ref[...]
                    + xh_ref[...])


def _combine(outm_f, agg_f, xh, inv_col, suma_col, wb, bl_row):
    grid = (NP // BN,)
    return pl.pallas_call(
        _comb_body,
        grid=grid,
        in_specs=[
            pl.BlockSpec((BN, D), lambda i: (i, 0)),
            pl.BlockSpec((BN, 16), lambda i: (i, 0)),
            pl.BlockSpec((BN, D), lambda i: (i, 0)),
            pl.BlockSpec((BN, 1), lambda i: (i, 0)),
            pl.BlockSpec((BN, 1), lambda i: (i, 0)),
            pl.BlockSpec((16, D), lambda i: (0, 0)),
            pl.BlockSpec((1, D), lambda i: (0, 0)),
        ],
        out_specs=pl.BlockSpec((BN, D), lambda i: (i, 0)),
        out_shape=jax.ShapeDtypeStruct((NP, D), jnp.float32),
    )(outm_f, agg_f, xh, inv_col, suma_col, wb, bl_row)


def kernel(x, edge_index, node_type, edge_type, edge_attr, hetero_W,
           hetero_b, edge_type_emb_table, W_edge_attr, W_att, W_lin, b_lin):
    f32 = jnp.float32
    x_p = jnp.pad(x, ((0, NP - N), (0, 0)))
    nt_p = jnp.pad(node_type, (0, NP - N)).reshape(NP, 1)
    wa_i = W_att[:D, :1]
    wa_j = W_att[D:2 * D, :1]
    wa_a = W_att[2 * D + 16:, :1]
    wa_t = W_att[2 * D:2 * D + 16, 0]
    wl_top = W_lin[:D, :]
    wb = W_lin[D:, :]

    xh, m, sd2, ss2 = _node_stage(x_p, nt_p, hetero_W, hetero_b,
                                  wl_top, wa_i, wa_j)

    ete = edge_type_emb_table
    ete_l = jnp.where(ete >= 0, ete, 0.2 * ete)
    t5 = ete_l @ wa_t
    t5_pad = jnp.pad(t5, (0, 3)).astype(f32)

    et_p = edge_type.reshape(E, 1)
    ea, c2 = _edge_stage(edge_attr, et_p, W_edge_attr, wa_a, t5_pad)

    srcA = edge_index[0]
    dstA = edge_index[1]
    cA = c2.reshape(E)
    sd = sd2.reshape(NP)
    ss = ss2.reshape(NP)
    outm_p, agg_p, den_p = _sc_stage(srcA, dstA, cA, ea, m, sd, ss)

    outm_f = outm_p.reshape(NP, D)
    agg_f = agg_p.reshape(NP, 16)
    inv2, suma2 = _den_stage(den_p)
    inv_col = inv2.reshape(NP, 1)
    suma_col = suma2.reshape(NP, 1)
    bl_row = b_lin.reshape(1, D)
    out_full = _combine(outm_f, agg_f, xh, inv_col, suma_col, wb, bl_row)
    return out_full[:N]


# speedup vs baseline: 6.8646x; 1.5848x over previous
import functools

import jax
import jax.numpy as jnp
from jax import lax
from jax.experimental import pallas as pl
from jax.experimental.pallas import tpu as pltpu
from jax.experimental.pallas import tpu_sc as plsc

N = 10000
NP = 10240
E = 320000
D = 128
CH = 64
NCH = E // CH
NWORK = 32
HALF = NP // 2
KMAX = -(-NCH // 16)
BN = 2048
BE = 8000


def _node_body(x_ref, nt_ref, hw_ref, hb_ref, wl_ref, wai_ref, waj_ref,
               xh_ref, m_ref, sd_ref, ss_ref):
    xb = x_ref[...]
    nt = nt_ref[...]
    acc = jnp.zeros((BN, D), dtype=jnp.float32)
    for t in range(3):
        pt = jnp.dot(xb, hw_ref[t], preferred_element_type=jnp.float32)
        pt = pt + hb_ref[pl.ds(t, 1), :]
        acc = jnp.where(nt == t, pt, acc)
    xh_ref[...] = acc
    m_ref[...] = jnp.dot(acc, wl_ref[...], preferred_element_type=jnp.float32)
    sd_ref[...] = jnp.dot(acc, wai_ref[...], preferred_element_type=jnp.float32)
    ss_ref[...] = jnp.dot(acc, waj_ref[...], preferred_element_type=jnp.float32)


def _node_stage(x_p, nt_p, hetero_W, hetero_b, wl_top, wa_i, wa_j):
    grid = (NP // BN,)
    return pl.pallas_call(
        _node_body,
        grid=grid,
        in_specs=[
            pl.BlockSpec((BN, D), lambda i: (i, 0)),
            pl.BlockSpec((BN, 1), lambda i: (i, 0)),
            pl.BlockSpec((3, D, D), lambda i: (0, 0, 0)),
            pl.BlockSpec((3, D), lambda i: (0, 0)),
            pl.BlockSpec((D, D), lambda i: (0, 0)),
            pl.BlockSpec((D, 1), lambda i: (0, 0)),
            pl.BlockSpec((D, 1), lambda i: (0, 0)),
        ],
        out_specs=[
            pl.BlockSpec((BN, D), lambda i: (i, 0)),
            pl.BlockSpec((BN, D), lambda i: (i, 0)),
            pl.BlockSpec((BN, 1), lambda i: (i, 0)),
            pl.BlockSpec((BN, 1), lambda i: (i, 0)),
        ],
        out_shape=[
            jax.ShapeDtypeStruct((NP, D), jnp.float32),
            jax.ShapeDtypeStruct((NP, D), jnp.float32),
            jax.ShapeDtypeStruct((NP, 1), jnp.float32),
            jax.ShapeDtypeStruct((NP, 1), jnp.float32),
        ],
    )(x_p, nt_p, hetero_W, hetero_b, wl_top, wa_i, wa_j)


def _edge_body(eattr_ref, et_ref, wea_ref, waa_ref, t5_ref, ea_ref, c_ref):
    v = jnp.dot(eattr_ref[...], wea_ref[...], preferred_element_type=jnp.float32)
    eab = jnp.where(v >= 0, v, 0.2 * v)
    ea_ref[...] = eab
    cc = jnp.dot(eab, waa_ref[...], preferred_element_type=jnp.float32)
    et = et_ref[...]
    for t in range(5):
        cc = jnp.where(et == t, cc + t5_ref[t], cc)
    c_ref[...] = cc


def _edge_stage(edge_attr, et_p, W_ea, wa_a, t5_pad):
    grid = (E // BE,)
    return pl.pallas_call(
        _edge_body,
        grid=grid,
        in_specs=[
            pl.BlockSpec((BE, 16), lambda i: (i, 0)),
            pl.BlockSpec((BE, 1), lambda i: (i, 0)),
            pl.BlockSpec((16, 16), lambda i: (0, 0)),
            pl.BlockSpec((16, 1), lambda i: (0, 0)),
            pl.BlockSpec(memory_space=pltpu.SMEM),
        ],
        out_specs=[
            pl.BlockSpec((BE, 16), lambda i: (i, 0)),
            pl.BlockSpec((BE, 1), lambda i: (i, 0)),
        ],
        out_shape=[
            jax.ShapeDtypeStruct((E, 16), jnp.float32),
            jax.ShapeDtypeStruct((E, 1), jnp.float32),
        ],
    )(edge_attr, et_p, W_ea, wa_a, t5_pad)


def _sc_body(src_h, dst_h, c_h, ea_h, m_h, sd_h, ss_h,
             outm_o, agg_o, den_o,
             sd_v, ss_v, den_v, *bufs):
    (src0, dst0, c0, sg0, dl0, ex0, eal0, eas0, mv0,
     src1, dst1, c1, sg1, dl1, ex1, eal1, eas1, mv1,
     outm_sh, agg_sh,
     ls0, gs0, ms0, as0, ls1, gs1, ms1, as1) = bufs
    SRC = (src0, src1)
    DST = (dst0, dst1)
    CV = (c0, c1)
    SG = (sg0, sg1)
    DL = (dl0, dl1)
    EX = (ex0, ex1)
    EAL = (eal0, eal1)
    EAS = (eas0, eas1)
    MV = (mv0, mv1)
    LSEM = (ls0, ls1)
    GSEM = (gs0, gs1)
    MSEM = (ms0, ms1)
    ASEM = (as0, as1)

    cid = lax.axis_index("c")
    sid = lax.axis_index("s")
    wid = sid * 2 + cid
    lo = cid * HALF
    cnt = (NCH - sid + 15) // 16
    z16v = jnp.zeros((16,), jnp.float32)

    def zrow(i, _):
        for kk in range(D // 16):
            mv0[i, pl.ds(kk * 16, 16)] = z16v
        eal0[i, :] = z16v
        return 0

    lax.fori_loop(0, CH, zrow, 0)

    def zden(i, _):
        den_v[i, :] = z16v
        return 0

    lax.fori_loop(0, NP // 16, zden, 0)

    rpm = HALF // 16
    for j in range(rpm // CH):
        r0 = sid * rpm + j * CH
        pltpu.sync_copy(mv0, outm_sh.at[pl.ds(r0, CH), :])
        pltpu.sync_copy(eal0, agg_sh.at[pl.ds(r0, CH), :])
    pltpu.sync_copy(sd_h, sd_v)
    pltpu.sync_copy(ss_h, ss_v)
    plsc.subcore_barrier()

    def fire_loads(k, b):
        base = (sid + k * 16) * CH
        pltpu.async_copy(src_h.at[pl.ds(base, CH)], SRC[b], LSEM[b])
        pltpu.async_copy(dst_h.at[pl.ds(base, CH)], DST[b], LSEM[b])
        pltpu.async_copy(c_h.at[pl.ds(base, CH)], CV[b], LSEM[b])
        pltpu.async_copy(ea_h.at[pl.ds(base, CH), :], EAL[b], LSEM[b])

    def wait_loads(k, b):
        base = (sid + k * 16) * CH
        pltpu.make_async_copy(src_h.at[pl.ds(base, CH)], SRC[b],
                              LSEM[b]).wait()
        pltpu.make_async_copy(dst_h.at[pl.ds(base, CH)], DST[b],
                              LSEM[b]).wait()
        pltpu.make_async_copy(c_h.at[pl.ds(base, CH)], CV[b],
                              LSEM[b]).wait()
        pltpu.make_async_copy(ea_h.at[pl.ds(base, CH), :], EAL[b],
                              LSEM[b]).wait()

    fire_loads(0, 0)

    def step(k, b):
        prev = 1 - b

        @pl.when((k >= 2) & (k - 2 < cnt))
        def _():
            pltpu.make_async_copy(
                MV[b], outm_sh.at[plsc.Indices(DL[b], ignored_value=-1)],
                MSEM[b]).wait()
            pltpu.make_async_copy(
                EAS[b], agg_sh.at[plsc.Indices(DL[b], ignored_value=-1)],
                ASEM[b]).wait()

        @pl.when(k < cnt)
        def _():
            wait_loads(k, b)
            for g in range(CH // 16):
                sl = pl.ds(g * 16, 16)
                s16 = SRC[b][sl]
                d16 = DST[b][sl]
                dloc = d16 - lo
                inh = (dloc >= 0) & (dloc < HALF)
                neg1 = jnp.zeros((16,), jnp.int32) - 1
                SG[b][sl] = jnp.where(inh, s16, neg1)
                DL[b][sl] = jnp.where(inh, dloc, neg1)
                z = (plsc.load_gather(sd_v, [d16]) +
                     plsc.load_gather(ss_v, [s16]) + CV[b][sl])
                z = jnp.where(z >= 0, z, 0.2 * z)
                ex = jnp.exp(z)
                EX[b][sl] = ex
                plsc.addupdate_scatter(
                    den_v, [lax.shift_right_logical(d16, 4),
                            lax.bitwise_and(d16, 15)], ex, mask=inh)
            pltpu.async_copy(
                m_h.at[plsc.Indices(SG[b], ignored_value=-1)], MV[b],
                GSEM[b])

            def ea_row(i, _):
                s = plsc.load_gather(EX[b], [jnp.zeros((16,), jnp.int32) + i])
                EAS[b][i, :] = EAL[b][i, :] * s
                return 0

            lax.fori_loop(0, CH, ea_row, 0)
            pltpu.async_copy(
                EAS[b], agg_sh.at[plsc.Indices(DL[b], ignored_value=-1)],
                ASEM[b], add=True)

        @pl.when(k + 1 < cnt)
        def _():
            fire_loads(k + 1, prev)

        @pl.when((k >= 1) & (k - 1 < cnt))
        def _():
            pltpu.make_async_copy(
                m_h.at[plsc.Indices(SG[prev], ignored_value=-1)], MV[prev],
                GSEM[prev]).wait()

            def m_row(i, _):
                s = plsc.load_gather(EX[prev],
                                     [jnp.zeros((16,), jnp.int32) + i])
                for kk in range(D // 16):
                    rsl = pl.ds(kk * 16, 16)
                    MV[prev][i, rsl] = MV[prev][i, rsl] * s
                return 0

            lax.fori_loop(0, CH, m_row, 0)
            pltpu.async_copy(
                MV[prev], outm_sh.at[plsc.Indices(DL[prev],
                                                  ignored_value=-1)],
                MSEM[prev], add=True)

    def pair_body(kk, _):
        step(2 * kk, 0)
        step(2 * kk + 1, 1)
        return 0

    lax.fori_loop(0, (KMAX + 3) // 2, pair_body, 0)

    pltpu.sync_copy(den_v, den_o.at[wid])
    plsc.subcore_barrier()

    for j in range(rpm // CH):
        r0 = sid * rpm + j * CH
        pltpu.sync_copy(outm_sh.at[pl.ds(r0, CH), :], mv0)
        pltpu.sync_copy(mv0, outm_o.at[cid, pl.ds(r0, CH), :])
        pltpu.sync_copy(agg_sh.at[pl.ds(r0, CH), :], eal0)
        pltpu.sync_copy(eal0, agg_o.at[cid, pl.ds(r0, CH), :])


def _sc_stage(srcA, dstA, cA, ea, m, sd, ss):
    mesh = plsc.VectorSubcoreMesh(core_axis_name="c", subcore_axis_name="s")
    fn = pl.kernel(
        _sc_body,
        out_type=[
            jax.ShapeDtypeStruct((2, HALF, D), jnp.float32),
            jax.ShapeDtypeStruct((2, HALF, 16), jnp.float32),
            jax.ShapeDtypeStruct((NWORK, NP // 16, 16), jnp.float32),
        ],
        mesh=mesh,
        compiler_params=pltpu.CompilerParams(
            needs_layout_passes=False, use_tc_tiling_on_sc=False),
        scratch_types=[
            pltpu.VMEM((NP,), jnp.float32),
            pltpu.VMEM((NP,), jnp.float32),
            pltpu.VMEM((NP // 16, 16), jnp.float32),
        ] + 2 * [
            pltpu.VMEM((CH,), jnp.int32),
            pltpu.VMEM((CH,), jnp.int32),
            pltpu.VMEM((CH,), jnp.float32),
            pltpu.VMEM((CH,), jnp.int32),
            pltpu.VMEM((CH,), jnp.int32),
            pltpu.VMEM((CH,), jnp.float32),
            pltpu.VMEM((CH, 16), jnp.float32),
            pltpu.VMEM((CH, 16), jnp.float32),
            pltpu.VMEM((CH, D), jnp.float32),
        ] + [
            pltpu.VMEM_SHARED((HALF, D), jnp.float32),
            pltpu.VMEM_SHARED((HALF, 16), jnp.float32),
        ] + 8 * [pltpu.SemaphoreType.DMA],
    )
    return fn(srcA, dstA, cA, ea, m, sd, ss)


def _den_body(den_ref, inv_ref, suma_ref):
    dsum = jnp.sum(den_ref[...], axis=0)
    inv = 1.0 / (dsum + 1e-16)
    inv_ref[...] = inv
    suma_ref[...] = dsum * inv


def _den_stage(den_p):
    return pl.pallas_call(
        _den_body,
        out_shape=[
            jax.ShapeDtypeStruct((NP // 16, 16), jnp.float32),
            jax.ShapeDtypeStruct((NP // 16, 16), jnp.float32),
        ],
    )(den_p)


def _comb_body(outm_ref, agg_ref, xh_ref, inv_ref, suma_ref, wb_ref, bl_ref,
               out_ref):
    core = outm_ref[...] + jnp.dot(
        agg_ref[...], wb_ref[...], preferred_element_type=jnp.float32)
    out_ref[...] = (core * inv_ref[...] + suma_ref[...] * bl_ref[...]
                    + xh_ref[...])


def _combine(outm_f, agg_f, xh, inv_col, suma_col, wb, bl_row):
    grid = (NP // BN,)
    return pl.pallas_call(
        _comb_body,
        grid=grid,
        in_specs=[
            pl.BlockSpec((BN, D), lambda i: (i, 0)),
            pl.BlockSpec((BN, 16), lambda i: (i, 0)),
            pl.BlockSpec((BN, D), lambda i: (i, 0)),
            pl.BlockSpec((BN, 1), lambda i: (i, 0)),
            pl.BlockSpec((BN, 1), lambda i: (i, 0)),
            pl.BlockSpec((16, D), lambda i: (0, 0)),
            pl.BlockSpec((1, D), lambda i: (0, 0)),
        ],
        out_specs=pl.BlockSpec((BN, D), lambda i: (i, 0)),
        out_shape=jax.ShapeDtypeStruct((NP, D), jnp.float32),
    )(outm_f, agg_f, xh, inv_col, suma_col, wb, bl_row)


def kernel(x, edge_index, node_type, edge_type, edge_attr, hetero_W,
           hetero_b, edge_type_emb_table, W_edge_attr, W_att, W_lin, b_lin):
    f32 = jnp.float32
    x_p = jnp.pad(x, ((0, NP - N), (0, 0)))
    nt_p = jnp.pad(node_type, (0, NP - N)).reshape(NP, 1)
    wa_i = W_att[:D, :1]
    wa_j = W_att[D:2 * D, :1]
    wa_a = W_att[2 * D + 16:, :1]
    wa_t = W_att[2 * D:2 * D + 16, 0]
    wl_top = W_lin[:D, :]
    wb = W_lin[D:, :]

    xh, m, sd2, ss2 = _node_stage(x_p, nt_p, hetero_W, hetero_b,
                                  wl_top, wa_i, wa_j)

    ete = edge_type_emb_table
    ete_l = jnp.where(ete >= 0, ete, 0.2 * ete)
    t5 = ete_l @ wa_t
    t5_pad = jnp.pad(t5, (0, 3)).astype(f32)

    et_p = edge_type.reshape(E, 1)
    ea, c2 = _edge_stage(edge_attr, et_p, W_edge_attr, wa_a, t5_pad)

    srcA = edge_index[0]
    dstA = edge_index[1]
    cA = c2.reshape(E)
    sd = sd2.reshape(NP)
    ss = ss2.reshape(NP)
    outm_p, agg_p, den_p = _sc_stage(srcA, dstA, cA, ea, m, sd, ss)

    outm_f = outm_p.reshape(NP, D)
    agg_f = agg_p.reshape(NP, 16)
    inv2, suma2 = _den_stage(den_p)
    inv_col = inv2.reshape(NP, 1)
    suma_col = suma2.reshape(NP, 1)
    bl_row = b_lin.reshape(1, D)
    out_full = _combine(outm_f, agg_f, xh, inv_col, suma_col, wb, bl_row)
    return out_full[:N]
